# Initial kernel scaffold; baseline (speedup 1.0000x reference)
import jax
import jax.numpy as jnp
from jax.experimental import pallas as pl


def kernel(x, edge_index, edge_values):
    raise NotImplementedError("write your pallas kernel here")


# speedup vs baseline: 4.4237x; 4.4237x over previous
import functools

import jax
import jax.numpy as jnp
from jax import lax
from jax.experimental import pallas as pl
from jax.experimental.pallas import tpu as pltpu
from jax.experimental.pallas import tpu_sc as plsc

N = 10000
D = 128
E = 320000
MIN_NORM = 1e-15
EPS = 4e-3

NC = 2
NS = 16
NW = NC * NS
EPW = E // NW
CH = 80
NCHUNK = EPW // CH
STRIPE = 632
STRIPE_LAST = N - (NS - 1) * STRIPE
LANES = 16
DSTEPS = D // LANES


def _logmap0_body(x_ref, o_ref):
    x = x_ref[...]
    nrm = jnp.maximum(jnp.sqrt(jnp.sum(x * x, axis=1, keepdims=True)), MIN_NORM)
    t = jnp.clip(nrm, -1.0 + 1e-7, 1.0 - 1e-7)
    at = 0.5 * (jnp.log1p(t) - jnp.log1p(-t))
    o_ref[...] = x / nrm * at


def _post_body(p_ref, o_ref):
    s = p_ref[0] + p_ref[1]
    nrm = jnp.maximum(jnp.sqrt(jnp.sum(s * s, axis=1, keepdims=True)), MIN_NORM)
    y = jnp.tanh(nrm) * s / nrm
    n2 = jnp.maximum(jnp.sqrt(jnp.sum(y * y, axis=1, keepdims=True)), MIN_NORM)
    maxnorm = 1.0 - EPS
    o_ref[...] = jnp.where(n2 > maxnorm, y / n2 * maxnorm, y)


def _logmap0(x):
    return pl.pallas_call(
        _logmap0_body,
        grid=(10,),
        in_specs=[pl.BlockSpec((N // 10, D), lambda i: (i, 0))],
        out_specs=pl.BlockSpec((N // 10, D), lambda i: (i, 0)),
        out_shape=jax.ShapeDtypeStruct((N, D), jnp.float32),
    )(x)


def _postmap(p):
    return pl.pallas_call(
        _post_body,
        grid=(10,),
        in_specs=[pl.BlockSpec((NC, N // 10, D), lambda i: (0, i, 0))],
        out_specs=pl.BlockSpec((N // 10, D), lambda i: (i, 0)),
        out_shape=jax.ShapeDtypeStruct((N, D), jnp.float32),
    )(p)


def _spmm_sc(xt, src, dst, vals, zeros):
    mesh = plsc.VectorSubcoreMesh(core_axis_name="c", subcore_axis_name="s")

    @functools.partial(
        pl.kernel,
        out_type=jax.ShapeDtypeStruct((NC, N, D), jnp.float32),
        mesh=mesh,
        scratch_types=[
            pltpu.VMEM((CH,), jnp.int32),
            pltpu.VMEM((CH,), jnp.int32),
            pltpu.VMEM((CH,), jnp.float32),
            pltpu.VMEM((CH, D), jnp.float32),
            pltpu.VMEM_SHARED((N, D), jnp.float32),
        ],
    )
    def k(xt_hbm, src_hbm, dst_hbm, vals_hbm, zeros_hbm, out_hbm,
          src_c, dst_c, vals_c, buf, acc):
        cid = lax.axis_index("c")
        sid = lax.axis_index("s")
        wid = cid * NS + sid

        @pl.when(sid < NS - 1)
        def _():
            pltpu.sync_copy(zeros_hbm.at[pl.ds(sid * STRIPE, STRIPE)],
                            acc.at[pl.ds(sid * STRIPE, STRIPE)])

        @pl.when(sid == NS - 1)
        def _():
            pltpu.sync_copy(zeros_hbm.at[pl.ds((NS - 1) * STRIPE, STRIPE_LAST)],
                            acc.at[pl.ds((NS - 1) * STRIPE, STRIPE_LAST)])

        plsc.subcore_barrier()

        def chunk(j, carry):
            base = wid * EPW + j * CH
            pltpu.sync_copy(src_hbm.at[pl.ds(base, CH)], src_c)
            pltpu.sync_copy(dst_hbm.at[pl.ds(base, CH)], dst_c)
            pltpu.sync_copy(vals_hbm.at[pl.ds(base, CH)], vals_c)
            pltpu.sync_copy(xt_hbm.at[src_c], buf)

            def gbody(g, c2):
                vv = vals_c[pl.ds(g * LANES, LANES)]
                for t in range(LANES):
                    v = jnp.take_along_axis(
                        vv, jnp.full((LANES,), t, jnp.int32), axis=0)
                    e = g * LANES + t
                    for d in range(DSTEPS):
                        sl = pl.ds(d * LANES, LANES)
                        buf[e, sl] = buf[e, sl] * v
                return c2

            lax.fori_loop(0, CH // LANES, gbody, 0)
            pltpu.sync_copy(buf, acc.at[dst_c], add=True)
            return carry

        lax.fori_loop(0, NCHUNK, chunk, 0)
        plsc.subcore_barrier()

        @pl.when(sid < NS - 1)
        def _():
            pltpu.sync_copy(acc.at[pl.ds(sid * STRIPE, STRIPE)],
                            out_hbm.at[cid, pl.ds(sid * STRIPE, STRIPE)])

        @pl.when(sid == NS - 1)
        def _():
            pltpu.sync_copy(
                acc.at[pl.ds((NS - 1) * STRIPE, STRIPE_LAST)],
                out_hbm.at[cid, pl.ds((NS - 1) * STRIPE, STRIPE_LAST)])

    return k(xt, src, dst, vals, zeros)


def kernel(x, edge_index, edge_values):
    xt = _logmap0(x)
    zeros = jnp.zeros((N, D), jnp.float32)
    partial = _spmm_sc(xt, edge_index[1], edge_index[0], edge_values, zeros)
    return _postmap(partial)


# speedup vs baseline: 6.9846x; 1.5789x over previous
import functools

import jax
import jax.numpy as jnp
from jax import lax
from jax.experimental import pallas as pl
from jax.experimental.pallas import tpu as pltpu
from jax.experimental.pallas import tpu_sc as plsc

N = 10000
D = 128
E = 320000
MIN_NORM = 1e-15
EPS = 4e-3

NC = 2
NS = 16
NW = NC * NS
EPW = E // NW
CH = 80
NCHUNK = EPW // CH
STRIPE = 632
STRIPE_LAST = N - (NS - 1) * STRIPE
LANES = 16
DSTEPS = D // LANES


def _logmap0_body(x_ref, o_ref):
    x = x_ref[...]
    nrm = jnp.maximum(jnp.sqrt(jnp.sum(x * x, axis=1, keepdims=True)), MIN_NORM)
    t = jnp.clip(nrm, -1.0 + 1e-7, 1.0 - 1e-7)
    at = 0.5 * (jnp.log1p(t) - jnp.log1p(-t))
    o_ref[...] = x / nrm * at


def _post_body(p_ref, o_ref):
    s = p_ref[0] + p_ref[1]
    nrm = jnp.maximum(jnp.sqrt(jnp.sum(s * s, axis=1, keepdims=True)), MIN_NORM)
    y = jnp.tanh(nrm) * s / nrm
    n2 = jnp.maximum(jnp.sqrt(jnp.sum(y * y, axis=1, keepdims=True)), MIN_NORM)
    maxnorm = 1.0 - EPS
    o_ref[...] = jnp.where(n2 > maxnorm, y / n2 * maxnorm, y)


def _logmap0(x):
    return pl.pallas_call(
        _logmap0_body,
        grid=(10,),
        in_specs=[pl.BlockSpec((N // 10, D), lambda i: (i, 0))],
        out_specs=pl.BlockSpec((N // 10, D), lambda i: (i, 0)),
        out_shape=jax.ShapeDtypeStruct((N, D), jnp.float32),
    )(x)


def _postmap(p):
    return pl.pallas_call(
        _post_body,
        grid=(10,),
        in_specs=[pl.BlockSpec((NC, N // 10, D), lambda i: (0, i, 0))],
        out_specs=pl.BlockSpec((N // 10, D), lambda i: (i, 0)),
        out_shape=jax.ShapeDtypeStruct((N, D), jnp.float32),
    )(p)


NBUF = 3


def _spmm_sc(xt, edata, vals, zeros):
    mesh = plsc.VectorSubcoreMesh(core_axis_name="c", subcore_axis_name="s")

    @functools.partial(
        pl.kernel,
        out_type=jax.ShapeDtypeStruct((NC, N, D), jnp.float32),
        mesh=mesh,
        scratch_types=[
            [pltpu.VMEM((2, CH), jnp.int32) for _ in range(NBUF)],
            [pltpu.VMEM((CH,), jnp.float32) for _ in range(NBUF)],
            [pltpu.VMEM((CH, D), jnp.float32) for _ in range(NBUF)],
            [pltpu.SemaphoreType.DMA for _ in range(NBUF)],
            [pltpu.SemaphoreType.DMA for _ in range(NBUF)],
            pltpu.VMEM_SHARED((N, D), jnp.float32),
        ],
    )
    def k(xt_hbm, edata_hbm, vals_hbm, zeros_hbm, out_hbm,
          ebuf, vbuf, rbuf, gsem, asem, acc):
        cid = lax.axis_index("c")
        sid = lax.axis_index("s")
        wid = cid * NS + sid

        @pl.when(sid < NS - 1)
        def _():
            pltpu.sync_copy(zeros_hbm.at[pl.ds(sid * STRIPE, STRIPE)],
                            acc.at[pl.ds(sid * STRIPE, STRIPE)])

        @pl.when(sid == NS - 1)
        def _():
            pltpu.sync_copy(zeros_hbm.at[pl.ds((NS - 1) * STRIPE, STRIPE_LAST)],
                            acc.at[pl.ds((NS - 1) * STRIPE, STRIPE_LAST)])

        plsc.subcore_barrier()

        def start_gather(j, b):
            pltpu.sync_copy(edata_hbm.at[wid * NCHUNK + j], ebuf[b])
            pltpu.sync_copy(vals_hbm.at[pl.ds((wid * NCHUNK + j) * CH, CH)],
                            vbuf[b])
            pltpu.async_copy(xt_hbm.at[ebuf[b].at[0]], rbuf[b], gsem[b])

        def wait_gather(b):
            pltpu.make_async_copy(xt_hbm.at[ebuf[b].at[0]], rbuf[b],
                                  gsem[b]).wait()

        def start_scatter(b):
            pltpu.async_copy(rbuf[b], acc.at[ebuf[b].at[1]], asem[b], add=True)

        def wait_scatter(b):
            pltpu.make_async_copy(rbuf[b], acc.at[ebuf[b].at[1]],
                                  asem[b]).wait()

        def scale(b):
            buf = rbuf[b]

            def gbody(g, c2):
                vv = vbuf[b][pl.ds(g * LANES, LANES)]
                for t in range(LANES):
                    v = jnp.take_along_axis(
                        vv, jnp.full((LANES,), t, jnp.int32), axis=0)
                    e = g * LANES + t
                    for d in range(DSTEPS):
                        sl = pl.ds(d * LANES, LANES)
                        buf[e, sl] = buf[e, sl] * v
                return c2

            lax.fori_loop(0, CH // LANES, gbody, 0)

        start_gather(0, 0)
        start_gather(1, 1)

        def triple(kk, carry):
            for b in range(NBUF):
                j = kk * NBUF + b
                bn = (b + 2) % NBUF

                @pl.when(j > 0)
                def _():
                    wait_scatter(bn)

                start_gather(j + 2, bn)

                wait_gather(b)
                scale(b)
                start_scatter(b)
            return carry

        lax.fori_loop(0, (NCHUNK - 2) // NBUF, triple, 0)

        for j in range(NCHUNK - 2, NCHUNK):
            b = j % NBUF
            wait_gather(b)
            scale(b)
            start_scatter(b)
        for j in range(NCHUNK - 3, NCHUNK):
            wait_scatter(j % NBUF)
        plsc.subcore_barrier()

        @pl.when(sid < NS - 1)
        def _():
            pltpu.sync_copy(acc.at[pl.ds(sid * STRIPE, STRIPE)],
                            out_hbm.at[cid, pl.ds(sid * STRIPE, STRIPE)])

        @pl.when(sid == NS - 1)
        def _():
            pltpu.sync_copy(
                acc.at[pl.ds((NS - 1) * STRIPE, STRIPE_LAST)],
                out_hbm.at[cid, pl.ds((NS - 1) * STRIPE, STRIPE_LAST)])

    return k(xt, edata, vals, zeros)


def kernel(x, edge_index, edge_values):
    xt = _logmap0(x)
    src = edge_index[1].reshape(NW * NCHUNK, CH)
    dst = edge_index[0].reshape(NW * NCHUNK, CH)
    edata = jnp.stack([src, dst], axis=1)
    zeros = jnp.zeros((N, D), jnp.float32)
    partial = _spmm_sc(xt, edata, edge_values, zeros)
    return _postmap(partial)
